# baseline (speedup 1.0000x reference)
import functools

import jax
import jax.numpy as jnp
from jax import lax
from jax.experimental import pallas as pl
from jax.experimental.pallas import tpu as pltpu
from jax.experimental.pallas import tpu_sc as plsc

_B = 16384
_D = 64
_NPOS = 20
_NNEG = 50
_NCTX = _NPOS + _NNEG
_PAD = 128
_NC = 2
_NS = 16
_NW = _NC * _NS
_BPW = _B // _NW
_C = 4
_NCHUNK = _BPW // _C
_NG = 5
_RST = 72
_ROWS = 296
_CPW = _C * _PAD
_FL = 16
_SCW = _FL * _CPW

_mesh = plsc.VectorSubcoreMesh(core_axis_name="c", subcore_axis_name="s")


@functools.partial(
    pl.kernel,
    mesh=_mesh,
    out_type=jax.ShapeDtypeStruct((_B * _PAD,), jnp.float32),
    scratch_types=[
        pltpu.VMEM((2, _C, _PAD), jnp.int32),
        pltpu.VMEM((2, _C, _PAD), jnp.float32),
        pltpu.VMEM((2, _ROWS, _D), jnp.float32),
        pltpu.VMEM((_C, 2 * _D), jnp.float32),
        pltpu.VMEM((_SCW,), jnp.float32),
        pltpu.SemaphoreType.DMA,
        pltpu.SemaphoreType.DMA,
    ],
    compiler_params=pltpu.CompilerParams(
        needs_layout_passes=False, use_tc_tiling_on_sc=False),
)
def _scores_kernel(out_tab, idx, vin, out,
                   ibuf, vbuf, rows, v2, scores_acc, gsem, ssem):
    wid = lax.axis_index("s") * _NC + lax.axis_index("c")
    iota = lax.iota(jnp.int32, 16)
    base = wid * _BPW * _PAD

    def stage_pair(c, islot):
        src = pl.ds(wid * _BPW + c * _C, _C)
        return (pltpu.async_copy(idx.at[src, :], ibuf.at[islot], ssem),
                pltpu.async_copy(vin.at[src, :], vbuf.at[islot], ssem))

    def gather_cps(c, buf):
        ib = lax.rem(c, 2)
        return [pltpu.async_copy(
            out_tab.at[ibuf.at[ib, b, pl.ds(0, _RST)]],
            rows.at[buf, pl.ds(b * _RST, _RST)], gsem)
            for b in range(_C)]

    for cp in stage_pair(0, 0):
        cp.wait()
    stage_pair(1, 1)
    gather_cps(0, 0)

    def chunk_body(ci, carry):
        buf = lax.rem(ci, 2)
        nxt = 1 - buf

        @pl.when(ci + 1 < _NCHUNK)
        def _():
            nsrc = pl.ds(wid * _BPW + (ci + 1) * _C, _C)
            pltpu.make_async_copy(
                idx.at[nsrc, :], ibuf.at[lax.rem(ci + 1, 2)], ssem).wait()
            pltpu.make_async_copy(
                vin.at[nsrc, :], vbuf.at[lax.rem(ci + 1, 2)], ssem).wait()
            gather_cps(ci + 1, nxt)

        for b in range(_C):
            pltpu.make_async_copy(
                out_tab.at[ibuf.at[buf, b, pl.ds(0, _RST)]],
                rows.at[buf, pl.ds(b * _RST, _RST)], gsem).wait()

        for b in range(_C):
            for k in range(4):
                c = vbuf[buf, b, pl.ds(k * 16, 16)]
                v2[b, pl.ds(k * 16, 16)] = c
                v2[b, pl.ds(_D + k * 16, 16)] = c

        @pl.when(ci + 2 < _NCHUNK)
        def _():
            stage_pair(ci + 2, lax.rem(ci, 2))

        soff = lax.rem(ci, _FL) * _CPW
        for b in range(_C):
            rbases = [(b * _RST + g * 16) + iota for g in range(_NG)]

            def d_body(dq, accs, b=b, rbases=rbases):
                accs = list(accs)
                for d4 in range(4):
                    d = dq * 4 + d4
                    civd = lax.bitwise_and(iota + d, jnp.int32(_D - 1))
                    vv = v2[b, pl.ds(d, 16)]
                    for g in range(_NG):
                        u = plsc.load_gather(rows.at[buf],
                                             [rbases[g], civd])
                        accs[g] = accs[g] + u * vv
                return tuple(accs)

            accs = lax.fori_loop(
                0, _D // 4, d_body,
                tuple([jnp.zeros((16,), jnp.float32)] * _NG))
            for g in range(_NG):
                scores_acc[pl.ds(soff + b * _PAD + g * 16, 16)] = accs[g]

        @pl.when(lax.rem(ci, _FL) == _FL - 1)
        def _():
            pltpu.sync_copy(
                scores_acc,
                out.at[pl.ds(base + (ci // _FL) * _SCW, _SCW)])

        return carry

    lax.fori_loop(0, _NCHUNK, chunk_body, 0)


def kernel(target, pos_context, neg_context, in_table, out_table):
    zpad = jnp.zeros((_B, _PAD - _NCTX), jnp.int32)
    idx = jnp.concatenate(
        [pos_context.astype(jnp.int32), neg_context.astype(jnp.int32),
         zpad], axis=1)
    v = jnp.take(in_table, target, axis=0)
    vp = jnp.concatenate(
        [v, jnp.zeros((_B, _PAD - _D), jnp.float32)], axis=1)
    scores = _scores_kernel(out_table, idx, vp)
    s = scores.reshape(_B, _PAD)
    return s[:, :_NPOS], s[:, _NPOS:_NCTX]


# speedup vs baseline: 1.5959x; 1.5959x over previous
import functools

import jax
import jax.numpy as jnp
from jax import lax
from jax.experimental import pallas as pl
from jax.experimental.pallas import tpu as pltpu
from jax.experimental.pallas import tpu_sc as plsc

_B = 16384
_D = 64
_NPOS = 20
_NNEG = 50
_NCTX = _NPOS + _NNEG
_PAD = 128
_NC = 2
_NS = 16
_NW = _NC * _NS
_BPW = _B // _NW
_C = 4
_NCHUNK = _BPW // _C
_NG = 5
_ROWS = 296
_CPW = _C * _PAD
_FL = 16
_SCW = _FL * _CPW

_mesh = plsc.VectorSubcoreMesh(core_axis_name="c", subcore_axis_name="s")


@functools.partial(
    pl.kernel,
    mesh=_mesh,
    out_type=jax.ShapeDtypeStruct((_B * _PAD,), jnp.float32),
    scratch_types=[
        pltpu.VMEM((2 * _CPW,), jnp.int32),
        pltpu.VMEM((2 * _CPW,), jnp.float32),
        pltpu.VMEM((2, _ROWS, _D), jnp.float32),
        pltpu.VMEM((_C, 2 * _D), jnp.float32),
        pltpu.VMEM((_SCW,), jnp.float32),
        pltpu.SemaphoreType.DMA,
        pltpu.SemaphoreType.DMA,
    ],
    compiler_params=pltpu.CompilerParams(
        needs_layout_passes=False, use_tc_tiling_on_sc=False),
)
def _scores_kernel(out_tab, idx, vin, out,
                   ibuf, vbuf, rows, v2, scores_acc, gsem, ssem):
    wid = lax.axis_index("s") * _NC + lax.axis_index("c")
    iota = lax.iota(jnp.int32, 16)
    base = wid * _BPW * _PAD

    def stage_pair(c, islot):
        src = pl.ds(base + c * _CPW, _CPW)
        return (pltpu.async_copy(idx.at[src],
                                 ibuf.at[pl.ds(islot * _CPW, _CPW)], ssem),
                pltpu.async_copy(vin.at[src],
                                 vbuf.at[pl.ds(islot * _CPW, _CPW)], ssem))

    def gather_cps(c, buf):
        ib = lax.rem(c, 2) * _CPW
        return [pltpu.async_copy(
            out_tab.at[ibuf.at[pl.ds(ib + b * _PAD, _NCTX)]],
            rows.at[buf, pl.ds(b * _NCTX, _NCTX)], gsem)
            for b in range(_C)]

    for cp in stage_pair(0, 0):
        cp.wait()
    stage_pair(1, 1)
    gather_cps(0, 0)

    def chunk_body(ci, carry):
        buf = lax.rem(ci, 2)
        nxt = 1 - buf

        @pl.when(ci + 1 < _NCHUNK)
        def _():
            pltpu.make_async_copy(
                idx.at[pl.ds(base + (ci + 1) * _CPW, _CPW)],
                ibuf.at[pl.ds(lax.rem(ci + 1, 2) * _CPW, _CPW)],
                ssem).wait()
            pltpu.make_async_copy(
                vin.at[pl.ds(base + (ci + 1) * _CPW, _CPW)],
                vbuf.at[pl.ds(lax.rem(ci + 1, 2) * _CPW, _CPW)],
                ssem).wait()
            gather_cps(ci + 1, nxt)

        ib = lax.rem(ci, 2) * _CPW
        for b in range(_C):
            pltpu.make_async_copy(
                out_tab.at[ibuf.at[pl.ds(ib + b * _PAD, _NCTX)]],
                rows.at[buf, pl.ds(b * _NCTX, _NCTX)], gsem).wait()

        vb = lax.rem(ci, 2) * _CPW
        for b in range(_C):
            for k in range(4):
                c = vbuf[pl.ds(vb + b * _PAD + k * 16, 16)]
                v2[b, pl.ds(k * 16, 16)] = c
                v2[b, pl.ds(_D + k * 16, 16)] = c

        @pl.when(ci + 2 < _NCHUNK)
        def _():
            stage_pair(ci + 2, lax.rem(ci, 2))

        soff = lax.rem(ci, _FL) * _CPW
        for b in range(_C):
            rbases = [(b * _NCTX + g * 16) + iota for g in range(_NG)]

            def d_body(dq, accs, b=b, rbases=rbases):
                accs = list(accs)
                for d4 in range(4):
                    d = dq * 4 + d4
                    civd = lax.bitwise_and(iota + d, jnp.int32(_D - 1))
                    vv = v2[b, pl.ds(d, 16)]
                    for g in range(_NG):
                        u = plsc.load_gather(rows.at[buf],
                                             [rbases[g], civd])
                        accs[g] = accs[g] + u * vv
                return tuple(accs)

            accs = lax.fori_loop(
                0, _D // 4, d_body,
                tuple([jnp.zeros((16,), jnp.float32)] * _NG))
            for g in range(_NG):
                scores_acc[pl.ds(soff + b * _PAD + g * 16, 16)] = accs[g]

        @pl.when(lax.rem(ci, _FL) == _FL - 1)
        def _():
            pltpu.sync_copy(
                scores_acc,
                out.at[pl.ds(base + (ci // _FL) * _SCW, _SCW)])

        return carry

    lax.fori_loop(0, _NCHUNK, chunk_body, 0)


def kernel(target, pos_context, neg_context, in_table, out_table):
    zpad = jnp.zeros((_B, _PAD - _NCTX), jnp.int32)
    idx = jnp.concatenate(
        [pos_context.astype(jnp.int32), neg_context.astype(jnp.int32),
         zpad], axis=1).reshape(_B * _PAD)
    v = jnp.take(in_table, target, axis=0)
    vp = jnp.concatenate(
        [v, jnp.zeros((_B, _PAD - _D), jnp.float32)], axis=1)
    scores = _scores_kernel(out_table, idx, vp.reshape(_B * _PAD))
    s = scores.reshape(_B, _PAD)
    return s[:, :_NPOS], s[:, _NPOS:_NCTX]
